# baseline (speedup 1.0000x reference)
import functools

import jax
import jax.numpy as jnp
from jax.experimental import pallas as pl
from jax.experimental.pallas import tpu as pltpu

N = 16384
IN_C = 64
OUT_C = 128
K = 16
M = 4096
MK = M * K


_KQ = 128
_KC = 128
_NCHUNK = N // _KC


def _bf(a):
    return a.astype(jnp.bfloat16).astype(jnp.float32)


def _knn_kernel(posT_ref, pn_ref, spos_ref, qn_ref, out_ref, scr_ref):
    qx = spos_ref[:, 0:1]
    qy = spos_ref[:, 1:2]
    qz = spos_ref[:, 2:3]
    qn = qn_ref[...]
    qxb = _bf(qx)
    qyb = _bf(qy)
    qzb = _bf(qz)

    lane = jax.lax.broadcasted_iota(jnp.int32, (_KQ, _KC), 1)
    inf = jnp.float32(jnp.inf)

    pv = jnp.full((_KQ, 1), -jnp.inf, dtype=jnp.float32)
    pi = jnp.full((_KQ, 1), -1, dtype=jnp.int32)

    for r in range(K):
        def chunk_body(c, carry):
            accv, acci = carry
            base = c * _KC
            if r == 0:
                xpc = _bf(posT_ref[0:1, pl.ds(base, _KC)])
                ypc = _bf(posT_ref[1:2, pl.ds(base, _KC)])
                zpc = _bf(posT_ref[2:3, pl.ds(base, _KC)])
                pnc = pn_ref[0:1, pl.ds(base, _KC)]
                v = (qn + pnc) - 2.0 * (qxb * xpc + qyb * ypc + qzb * zpc)
                scr_ref[:, pl.ds(base, _KC)] = v
            else:
                v = scr_ref[:, pl.ds(base, _KC)]
            idx = lane + base
            elig = (v > pv) | ((v == pv) & (idx > pi))
            vm = jnp.where(elig, v, inf)
            take = vm < accv
            acci = jnp.where(take, idx, acci)
            accv = jnp.minimum(accv, vm)
            return (accv, acci)

        accv0 = jnp.full((_KQ, _KC), jnp.inf, dtype=jnp.float32)
        acci0 = jnp.zeros((_KQ, _KC), dtype=jnp.int32)
        accv, acci = jax.lax.fori_loop(0, _NCHUNK, chunk_body, (accv0, acci0))

        v, i = accv, acci
        half = _KC // 2
        while half >= 1:
            lv, li = v[:, :half], i[:, :half]
            rv, ri = v[:, half:2 * half], i[:, half:2 * half]
            take = (rv < lv) | ((rv == lv) & (ri < li))
            v = jnp.where(take, rv, lv)
            i = jnp.where(take, ri, li)
            half //= 2
        out_ref[:, r:r + 1] = i
        pv, pi = v, i


def _knn(pos, sub_pos):
    posT = pos.T
    pn = jnp.sum(pos ** 2, axis=1)[None, :]
    qn = jnp.sum(sub_pos ** 2, axis=1)[:, None]
    return pl.pallas_call(
        _knn_kernel,
        grid=(M // _KQ,),
        in_specs=[
            pl.BlockSpec((3, N), lambda i: (0, 0)),
            pl.BlockSpec((1, N), lambda i: (0, 0)),
            pl.BlockSpec((_KQ, 3), lambda i: (i, 0)),
            pl.BlockSpec((_KQ, 1), lambda i: (i, 0)),
        ],
        out_specs=pl.BlockSpec((_KQ, K), lambda i: (i, 0)),
        out_shape=jax.ShapeDtypeStruct((M, K), jnp.int32),
        scratch_shapes=[pltpu.VMEM((_KQ, N), jnp.float32)],
    )(posT, pn, sub_pos, qn)


def _z_kernel(xp_ref, wt_ref, z_ref):
    z_ref[...] = jax.lax.dot(xp_ref[...], wt_ref[...],
                             precision=jax.lax.Precision.HIGHEST)


def _compute_z(xp, wt):
    TR = 2048
    return pl.pallas_call(
        _z_kernel,
        grid=(N // TR,),
        in_specs=[
            pl.BlockSpec((TR, xp.shape[1]), lambda i: (i, 0)),
            pl.BlockSpec((xp.shape[1], OUT_C), lambda i: (0, 0)),
        ],
        out_specs=pl.BlockSpec((TR, OUT_C), lambda i: (i, 0)),
        out_shape=jax.ShapeDtypeStruct((N, OUT_C), jnp.float32),
    )(xp, wt)


def _stats_kernel(zg_ref, mx_ref, s1_ref, s1sum_ref, s2sum_ref):
    step = pl.program_id(0)
    zt = zg_ref[...]
    z3 = zt.reshape(zt.shape[0] // K, K, OUT_C)
    s1 = z3.sum(axis=1)
    s2 = (z3 * z3).sum(axis=1)
    mx = z3.max(axis=1)
    mx_ref[...] = mx
    s1_ref[...] = s1
    ps1 = s1.sum(axis=0, keepdims=True)
    ps2 = s2.sum(axis=0, keepdims=True)

    @pl.when(step == 0)
    def _():
        s1sum_ref[...] = jnp.zeros_like(s1sum_ref)
        s2sum_ref[...] = jnp.zeros_like(s2sum_ref)

    s1sum_ref[...] += ps1
    s2sum_ref[...] += ps2


def _segment_stats(zg):
    TR = 4096
    SEG = TR // K
    return pl.pallas_call(
        _stats_kernel,
        grid=(MK // TR,),
        in_specs=[pl.BlockSpec((TR, OUT_C), lambda i: (i, 0))],
        out_specs=[
            pl.BlockSpec((SEG, OUT_C), lambda i: (i, 0)),
            pl.BlockSpec((SEG, OUT_C), lambda i: (i, 0)),
            pl.BlockSpec((1, OUT_C), lambda i: (0, 0)),
            pl.BlockSpec((1, OUT_C), lambda i: (0, 0)),
        ],
        out_shape=[
            jax.ShapeDtypeStruct((M, OUT_C), jnp.float32),
            jax.ShapeDtypeStruct((M, OUT_C), jnp.float32),
            jax.ShapeDtypeStruct((1, OUT_C), jnp.float32),
            jax.ShapeDtypeStruct((1, OUT_C), jnp.float32),
        ],
    )(zg)


def _finish_kernel(posm_ref, wpt_ref, mx_ref, s1_ref, s1sum_ref,
                   s2sum_ref, gamma_ref, beta_ref, out_ref):
    q = jax.lax.dot(posm_ref[...], wpt_ref[...],
                    precision=jax.lax.Precision.HIGHEST)
    s1 = s1_ref[...]
    qs = q.sum(axis=0, keepdims=True)
    mean = (s1sum_ref[...] - K * qs) / MK
    cross = (q * s1).sum(axis=0, keepdims=True)
    h2 = s2sum_ref[...] - 2.0 * cross + K * (q * q).sum(axis=0, keepdims=True)
    var = h2 / MK - mean * mean
    inv = jax.lax.rsqrt(var + 1e-5) * gamma_ref[...]
    out_ref[...] = jnp.maximum((mx_ref[...] - q - mean) * inv + beta_ref[...],
                               0.0)


def _finish(posm, wpt, mx, s1, s1sum, s2sum, gamma, beta):
    return pl.pallas_call(
        _finish_kernel,
        out_shape=jax.ShapeDtypeStruct((M, OUT_C), jnp.float32),
    )(posm, wpt, mx, s1, s1sum, s2sum, gamma.reshape(1, OUT_C),
      beta.reshape(1, OUT_C))


_FR = 128
_FC = N // _FR


def _fps_kernel(px_ref, py_ref, pz_ref, out_ref):
    px = px_ref[...]
    py = py_ref[...]
    pz = pz_ref[...]
    rows = jax.lax.broadcasted_iota(jnp.int32, (_FR, _FC), 0)
    cols = jax.lax.broadcasted_iota(jnp.int32, (_FR, _FC), 1)
    idx = rows * _FC + cols
    out_ref[0] = 0
    lx0 = px[0, 0]
    ly0 = py[0, 0]
    lz0 = pz[0, 0]
    dists0 = jnp.full((_FR, _FC), jnp.inf, dtype=jnp.float32)

    def body(i, carry):
        lx, ly, lz, dists = carry
        dx = px - lx
        dy = py - ly
        dz = pz - lz
        d = (dx * dx + dz * dz) + dy * dy
        dists = jnp.minimum(dists, d)
        mx = jnp.max(dists)
        cand = jnp.where(dists == mx, idx, jnp.int32(N))
        nxt = jnp.min(cand)
        out_ref[i] = nxt
        m = idx == nxt
        zero = jnp.float32(0.0)
        nlx = jnp.sum(jnp.where(m, px, zero))
        nly = jnp.sum(jnp.where(m, py, zero))
        nlz = jnp.sum(jnp.where(m, pz, zero))
        return (nlx, nly, nlz, dists)

    jax.lax.fori_loop(1, M, body, (lx0, ly0, lz0, dists0))


def _fps(pos):
    px = pos[:, 0].reshape(_FR, _FC)
    py = pos[:, 1].reshape(_FR, _FC)
    pz = pos[:, 2].reshape(_FR, _FC)
    return pl.pallas_call(
        _fps_kernel,
        in_specs=[
            pl.BlockSpec(memory_space=pltpu.VMEM),
            pl.BlockSpec(memory_space=pltpu.VMEM),
            pl.BlockSpec(memory_space=pltpu.VMEM),
        ],
        out_specs=pl.BlockSpec(memory_space=pltpu.SMEM),
        out_shape=jax.ShapeDtypeStruct((M,), jnp.int32),
    )(px, py, pz)


def kernel(x, pos, batch, W, gamma, beta):
    id_clusters = _fps(pos)
    sub_pos = pos[id_clusters]
    sub_batch = batch[id_clusters]

    nn = _knn(pos, sub_pos)
    col = nn.reshape(-1)

    xp = jnp.concatenate([pos, x], axis=1)
    z = _compute_z(xp, W.T)
    zg = z[col]

    mx, s1, s1sum, s2sum = _segment_stats(zg)
    x_out = _finish(pos[:M], W[:, :3].T, mx, s1, s1sum, s2sum, gamma, beta)
    return (x_out, sub_pos, sub_batch)


# speedup vs baseline: 1.9105x; 1.9105x over previous
import functools

import jax
import jax.numpy as jnp
from jax.experimental import pallas as pl
from jax.experimental.pallas import tpu as pltpu

N = 16384
IN_C = 64
OUT_C = 128
K = 16
M = 4096
MK = M * K


_KQ = 32
_KC = 128
_NCHUNK = N // _KC


def _bf(a):
    return a.astype(jnp.bfloat16).astype(jnp.float32)


_KU = 4


def _knn_kernel(posT_ref, pn_ref, spos_ref, qn_ref, out_ref, scr_ref):
    shape = (_KQ, _KC)
    lane = jax.lax.broadcasted_iota(jnp.int32, shape, 1)
    inf = jnp.float32(jnp.inf)

    qxb = jnp.broadcast_to(_bf(spos_ref[:, 0:1]), shape)
    qyb = jnp.broadcast_to(_bf(spos_ref[:, 1:2]), shape)
    qzb = jnp.broadcast_to(_bf(spos_ref[:, 2:3]), shape)
    qn = jnp.broadcast_to(qn_ref[...], shape)

    def fill_body(c, _):
        base = c * _KC
        xpc = _bf(posT_ref[0:1, pl.ds(base, _KC)])
        ypc = _bf(posT_ref[1:2, pl.ds(base, _KC)])
        zpc = _bf(posT_ref[2:3, pl.ds(base, _KC)])
        pnc = pn_ref[0:1, pl.ds(base, _KC)]
        v = (qn + pnc) - 2.0 * (qxb * xpc + qyb * ypc + qzb * zpc)
        scr_ref[:, pl.ds(base, _KC)] = v
        return 0

    jax.lax.fori_loop(0, _NCHUNK, fill_body, 0)

    pv = jnp.full(shape, -jnp.inf, dtype=jnp.float32)
    pi = jnp.full(shape, -1, dtype=jnp.int32)

    for r in range(K):
        def chunk_body(c, carry):
            av0, ai0, av1, ai1 = carry
            accs = [(av0, ai0), (av1, ai1)]
            for u in range(_KU):
                base = (c * _KU + u) * _KC
                v = scr_ref[:, pl.ds(base, _KC)]
                idx = lane + base
                elig = (v > pv) | ((v == pv) & (idx > pi))
                vm = jnp.where(elig, v, inf)
                accv, acci = accs[u % 2]
                take = vm < accv
                acci = jnp.where(take, idx, acci)
                accv = jnp.minimum(accv, vm)
                accs[u % 2] = (accv, acci)
            return (*accs[0], *accs[1])

        accv0 = jnp.full(shape, jnp.inf, dtype=jnp.float32)
        acci0 = jnp.zeros(shape, dtype=jnp.int32)
        av0, ai0, av1, ai1 = jax.lax.fori_loop(
            0, _NCHUNK // _KU, chunk_body, (accv0, acci0, accv0, acci0))
        take = (av1 < av0) | ((av1 == av0) & (ai1 < ai0))
        v = jnp.where(take, av1, av0)
        i = jnp.where(take, ai1, ai0)

        half = _KC // 2
        while half >= 1:
            lv, li = v[:, :half], i[:, :half]
            rv, ri = v[:, half:2 * half], i[:, half:2 * half]
            take = (rv < lv) | ((rv == lv) & (ri < li))
            v = jnp.where(take, rv, lv)
            i = jnp.where(take, ri, li)
            half //= 2
        out_ref[:, r:r + 1] = i
        pv = jnp.broadcast_to(v, shape)
        pi = jnp.broadcast_to(i, shape)


def _knn(pos, sub_pos):
    posT = pos.T
    pn = jnp.sum(pos ** 2, axis=1)[None, :]
    qn = jnp.sum(sub_pos ** 2, axis=1)[:, None]
    return pl.pallas_call(
        _knn_kernel,
        grid=(M // _KQ,),
        in_specs=[
            pl.BlockSpec((3, N), lambda i: (0, 0)),
            pl.BlockSpec((1, N), lambda i: (0, 0)),
            pl.BlockSpec((_KQ, 3), lambda i: (i, 0)),
            pl.BlockSpec((_KQ, 1), lambda i: (i, 0)),
        ],
        out_specs=pl.BlockSpec((_KQ, K), lambda i: (i, 0)),
        out_shape=jax.ShapeDtypeStruct((M, K), jnp.int32),
        scratch_shapes=[pltpu.VMEM((_KQ, N), jnp.float32)],
    )(posT, pn, sub_pos, qn)


def _z_kernel(xp_ref, wt_ref, z_ref):
    z_ref[...] = jax.lax.dot(xp_ref[...], wt_ref[...],
                             precision=jax.lax.Precision.HIGHEST)


def _compute_z(xp, wt):
    TR = 2048
    return pl.pallas_call(
        _z_kernel,
        grid=(N // TR,),
        in_specs=[
            pl.BlockSpec((TR, xp.shape[1]), lambda i: (i, 0)),
            pl.BlockSpec((xp.shape[1], OUT_C), lambda i: (0, 0)),
        ],
        out_specs=pl.BlockSpec((TR, OUT_C), lambda i: (i, 0)),
        out_shape=jax.ShapeDtypeStruct((N, OUT_C), jnp.float32),
    )(xp, wt)


def _stats_kernel(zg_ref, mx_ref, s1_ref, s1sum_ref, s2sum_ref):
    step = pl.program_id(0)
    zt = zg_ref[...]
    z3 = zt.reshape(zt.shape[0] // K, K, OUT_C)
    s1 = z3.sum(axis=1)
    s2 = (z3 * z3).sum(axis=1)
    mx = z3.max(axis=1)
    mx_ref[...] = mx
    s1_ref[...] = s1
    ps1 = s1.sum(axis=0, keepdims=True)
    ps2 = s2.sum(axis=0, keepdims=True)

    @pl.when(step == 0)
    def _():
        s1sum_ref[...] = jnp.zeros_like(s1sum_ref)
        s2sum_ref[...] = jnp.zeros_like(s2sum_ref)

    s1sum_ref[...] += ps1
    s2sum_ref[...] += ps2


def _segment_stats(zg):
    TR = 4096
    SEG = TR // K
    return pl.pallas_call(
        _stats_kernel,
        grid=(MK // TR,),
        in_specs=[pl.BlockSpec((TR, OUT_C), lambda i: (i, 0))],
        out_specs=[
            pl.BlockSpec((SEG, OUT_C), lambda i: (i, 0)),
            pl.BlockSpec((SEG, OUT_C), lambda i: (i, 0)),
            pl.BlockSpec((1, OUT_C), lambda i: (0, 0)),
            pl.BlockSpec((1, OUT_C), lambda i: (0, 0)),
        ],
        out_shape=[
            jax.ShapeDtypeStruct((M, OUT_C), jnp.float32),
            jax.ShapeDtypeStruct((M, OUT_C), jnp.float32),
            jax.ShapeDtypeStruct((1, OUT_C), jnp.float32),
            jax.ShapeDtypeStruct((1, OUT_C), jnp.float32),
        ],
    )(zg)


def _finish_kernel(posm_ref, wpt_ref, mx_ref, s1_ref, s1sum_ref,
                   s2sum_ref, gamma_ref, beta_ref, out_ref):
    q = jax.lax.dot(posm_ref[...], wpt_ref[...],
                    precision=jax.lax.Precision.HIGHEST)
    s1 = s1_ref[...]
    qs = q.sum(axis=0, keepdims=True)
    mean = (s1sum_ref[...] - K * qs) / MK
    cross = (q * s1).sum(axis=0, keepdims=True)
    h2 = s2sum_ref[...] - 2.0 * cross + K * (q * q).sum(axis=0, keepdims=True)
    var = h2 / MK - mean * mean
    inv = jax.lax.rsqrt(var + 1e-5) * gamma_ref[...]
    out_ref[...] = jnp.maximum((mx_ref[...] - q - mean) * inv + beta_ref[...],
                               0.0)


def _finish(posm, wpt, mx, s1, s1sum, s2sum, gamma, beta):
    return pl.pallas_call(
        _finish_kernel,
        out_shape=jax.ShapeDtypeStruct((M, OUT_C), jnp.float32),
    )(posm, wpt, mx, s1, s1sum, s2sum, gamma.reshape(1, OUT_C),
      beta.reshape(1, OUT_C))


_FR = 128
_FC = N // _FR


def _fps_kernel(px_ref, py_ref, pz_ref, out_ref):
    px = px_ref[...]
    py = py_ref[...]
    pz = pz_ref[...]
    rows = jax.lax.broadcasted_iota(jnp.int32, (_FR, _FC), 0)
    cols = jax.lax.broadcasted_iota(jnp.int32, (_FR, _FC), 1)
    idx = rows * _FC + cols
    out_ref[0] = 0
    lx0 = px[0, 0]
    ly0 = py[0, 0]
    lz0 = pz[0, 0]
    dists0 = jnp.full((_FR, _FC), jnp.inf, dtype=jnp.float32)

    def body(i, carry):
        lx, ly, lz, dists = carry
        dx = px - lx
        dy = py - ly
        dz = pz - lz
        d = (dx * dx + dz * dz) + dy * dy
        dists = jnp.minimum(dists, d)
        mx = jnp.max(dists)
        cand = jnp.where(dists == mx, idx, jnp.int32(N))
        nxt = jnp.min(cand)
        out_ref[i] = nxt
        m = idx == nxt
        zero = jnp.float32(0.0)
        nlx = jnp.sum(jnp.where(m, px, zero))
        nly = jnp.sum(jnp.where(m, py, zero))
        nlz = jnp.sum(jnp.where(m, pz, zero))
        return (nlx, nly, nlz, dists)

    jax.lax.fori_loop(1, M, body, (lx0, ly0, lz0, dists0))


def _fps(pos):
    px = pos[:, 0].reshape(_FR, _FC)
    py = pos[:, 1].reshape(_FR, _FC)
    pz = pos[:, 2].reshape(_FR, _FC)
    return pl.pallas_call(
        _fps_kernel,
        in_specs=[
            pl.BlockSpec(memory_space=pltpu.VMEM),
            pl.BlockSpec(memory_space=pltpu.VMEM),
            pl.BlockSpec(memory_space=pltpu.VMEM),
        ],
        out_specs=pl.BlockSpec(memory_space=pltpu.SMEM),
        out_shape=jax.ShapeDtypeStruct((M,), jnp.int32),
    )(px, py, pz)


def kernel(x, pos, batch, W, gamma, beta):
    id_clusters = _fps(pos)
    sub_pos = pos[id_clusters]
    sub_batch = batch[id_clusters]

    nn = _knn(pos, sub_pos)
    col = nn.reshape(-1)

    xp = jnp.concatenate([pos, x], axis=1)
    z = _compute_z(xp, W.T)
    zg = z[col]

    mx, s1, s1sum, s2sum = _segment_stats(zg)
    x_out = _finish(pos[:M], W[:, :3].T, mx, s1, s1sum, s2sum, gamma, beta)
    return (x_out, sub_pos, sub_batch)


# speedup vs baseline: 1.9944x; 1.0439x over previous
import functools

import jax
import jax.numpy as jnp
from jax.experimental import pallas as pl
from jax.experimental.pallas import tpu as pltpu

N = 16384
IN_C = 64
OUT_C = 128
K = 16
M = 4096
MK = M * K


_KQ = 32
_KC = 128
_NCHUNK = N // _KC


def _bf(a):
    return a.astype(jnp.bfloat16).astype(jnp.float32)


_KU = 8


def _knn_kernel(posT_ref, pn_ref, spos_ref, qn_ref, out_ref, scr_ref):
    shape = (_KQ, _KC)
    lane = jax.lax.broadcasted_iota(jnp.int32, shape, 1)
    inf = jnp.float32(jnp.inf)

    qxb = jnp.broadcast_to(_bf(spos_ref[:, 0:1]), shape)
    qyb = jnp.broadcast_to(_bf(spos_ref[:, 1:2]), shape)
    qzb = jnp.broadcast_to(_bf(spos_ref[:, 2:3]), shape)
    qn = jnp.broadcast_to(qn_ref[...], shape)

    def fill_body(c, _):
        base = c * _KC
        xpc = _bf(posT_ref[0:1, pl.ds(base, _KC)])
        ypc = _bf(posT_ref[1:2, pl.ds(base, _KC)])
        zpc = _bf(posT_ref[2:3, pl.ds(base, _KC)])
        pnc = pn_ref[0:1, pl.ds(base, _KC)]
        v = (qn + pnc) - 2.0 * (qxb * xpc + qyb * ypc + qzb * zpc)
        scr_ref[:, pl.ds(base, _KC)] = v
        return 0

    jax.lax.fori_loop(0, _NCHUNK, fill_body, 0)

    pv = jnp.full(shape, -jnp.inf, dtype=jnp.float32)
    pi = jnp.full(shape, -1, dtype=jnp.int32)

    for r in range(K):
        def chunk_body(c, carry):
            av0, ai0, av1, ai1 = carry
            accs = [(av0, ai0), (av1, ai1)]
            for u in range(_KU):
                base = (c * _KU + u) * _KC
                v = scr_ref[:, pl.ds(base, _KC)]
                idx = lane + base
                elig = (v > pv) | ((v == pv) & (idx > pi))
                vm = jnp.where(elig, v, inf)
                accv, acci = accs[u % 2]
                take = vm < accv
                acci = jnp.where(take, idx, acci)
                accv = jnp.minimum(accv, vm)
                accs[u % 2] = (accv, acci)
            return (*accs[0], *accs[1])

        accv0 = jnp.full(shape, jnp.inf, dtype=jnp.float32)
        acci0 = jnp.zeros(shape, dtype=jnp.int32)
        av0, ai0, av1, ai1 = jax.lax.fori_loop(
            0, _NCHUNK // _KU, chunk_body, (accv0, acci0, accv0, acci0))
        take = (av1 < av0) | ((av1 == av0) & (ai1 < ai0))
        v = jnp.where(take, av1, av0)
        i = jnp.where(take, ai1, ai0)

        half = _KC // 2
        while half >= 1:
            lv, li = v[:, :half], i[:, :half]
            rv, ri = v[:, half:2 * half], i[:, half:2 * half]
            take = (rv < lv) | ((rv == lv) & (ri < li))
            v = jnp.where(take, rv, lv)
            i = jnp.where(take, ri, li)
            half //= 2
        out_ref[:, r:r + 1] = i
        pv = jnp.broadcast_to(v, shape)
        pi = jnp.broadcast_to(i, shape)


def _knn(pos, sub_pos):
    posT = pos.T
    pn = jnp.sum(pos ** 2, axis=1)[None, :]
    qn = jnp.sum(sub_pos ** 2, axis=1)[:, None]
    return pl.pallas_call(
        _knn_kernel,
        grid=(M // _KQ,),
        in_specs=[
            pl.BlockSpec((3, N), lambda i: (0, 0)),
            pl.BlockSpec((1, N), lambda i: (0, 0)),
            pl.BlockSpec((_KQ, 3), lambda i: (i, 0)),
            pl.BlockSpec((_KQ, 1), lambda i: (i, 0)),
        ],
        out_specs=pl.BlockSpec((_KQ, K), lambda i: (i, 0)),
        out_shape=jax.ShapeDtypeStruct((M, K), jnp.int32),
        scratch_shapes=[pltpu.VMEM((_KQ, N), jnp.float32)],
    )(posT, pn, sub_pos, qn)


def _z_kernel(xp_ref, wt_ref, z_ref):
    z_ref[...] = jax.lax.dot(xp_ref[...], wt_ref[...],
                             precision=jax.lax.Precision.HIGHEST)


def _compute_z(xp, wt):
    TR = 2048
    return pl.pallas_call(
        _z_kernel,
        grid=(N // TR,),
        in_specs=[
            pl.BlockSpec((TR, xp.shape[1]), lambda i: (i, 0)),
            pl.BlockSpec((xp.shape[1], OUT_C), lambda i: (0, 0)),
        ],
        out_specs=pl.BlockSpec((TR, OUT_C), lambda i: (i, 0)),
        out_shape=jax.ShapeDtypeStruct((N, OUT_C), jnp.float32),
    )(xp, wt)


def _stats_kernel(zg_ref, mx_ref, s1_ref, s1sum_ref, s2sum_ref):
    step = pl.program_id(0)
    zt = zg_ref[...]
    z3 = zt.reshape(zt.shape[0] // K, K, OUT_C)
    s1 = z3.sum(axis=1)
    s2 = (z3 * z3).sum(axis=1)
    mx = z3.max(axis=1)
    mx_ref[...] = mx
    s1_ref[...] = s1
    ps1 = s1.sum(axis=0, keepdims=True)
    ps2 = s2.sum(axis=0, keepdims=True)

    @pl.when(step == 0)
    def _():
        s1sum_ref[...] = jnp.zeros_like(s1sum_ref)
        s2sum_ref[...] = jnp.zeros_like(s2sum_ref)

    s1sum_ref[...] += ps1
    s2sum_ref[...] += ps2


def _segment_stats(zg):
    TR = 4096
    SEG = TR // K
    return pl.pallas_call(
        _stats_kernel,
        grid=(MK // TR,),
        in_specs=[pl.BlockSpec((TR, OUT_C), lambda i: (i, 0))],
        out_specs=[
            pl.BlockSpec((SEG, OUT_C), lambda i: (i, 0)),
            pl.BlockSpec((SEG, OUT_C), lambda i: (i, 0)),
            pl.BlockSpec((1, OUT_C), lambda i: (0, 0)),
            pl.BlockSpec((1, OUT_C), lambda i: (0, 0)),
        ],
        out_shape=[
            jax.ShapeDtypeStruct((M, OUT_C), jnp.float32),
            jax.ShapeDtypeStruct((M, OUT_C), jnp.float32),
            jax.ShapeDtypeStruct((1, OUT_C), jnp.float32),
            jax.ShapeDtypeStruct((1, OUT_C), jnp.float32),
        ],
    )(zg)


def _finish_kernel(posm_ref, wpt_ref, mx_ref, s1_ref, s1sum_ref,
                   s2sum_ref, gamma_ref, beta_ref, out_ref):
    q = jax.lax.dot(posm_ref[...], wpt_ref[...],
                    precision=jax.lax.Precision.HIGHEST)
    s1 = s1_ref[...]
    qs = q.sum(axis=0, keepdims=True)
    mean = (s1sum_ref[...] - K * qs) / MK
    cross = (q * s1).sum(axis=0, keepdims=True)
    h2 = s2sum_ref[...] - 2.0 * cross + K * (q * q).sum(axis=0, keepdims=True)
    var = h2 / MK - mean * mean
    inv = jax.lax.rsqrt(var + 1e-5) * gamma_ref[...]
    out_ref[...] = jnp.maximum((mx_ref[...] - q - mean) * inv + beta_ref[...],
                               0.0)


def _finish(posm, wpt, mx, s1, s1sum, s2sum, gamma, beta):
    return pl.pallas_call(
        _finish_kernel,
        out_shape=jax.ShapeDtypeStruct((M, OUT_C), jnp.float32),
    )(posm, wpt, mx, s1, s1sum, s2sum, gamma.reshape(1, OUT_C),
      beta.reshape(1, OUT_C))


_FR = 128
_FC = N // _FR


def _fps_kernel(px_ref, py_ref, pz_ref, out_ref):
    px = px_ref[...]
    py = py_ref[...]
    pz = pz_ref[...]
    rows = jax.lax.broadcasted_iota(jnp.int32, (_FR, _FC), 0)
    cols = jax.lax.broadcasted_iota(jnp.int32, (_FR, _FC), 1)
    idx = rows * _FC + cols
    lane1 = jax.lax.broadcasted_iota(jnp.int32, (1, _FC), 1)
    out_ref[0] = 0
    lx0 = px[0, 0]
    ly0 = py[0, 0]
    lz0 = pz[0, 0]
    dists0 = jnp.full((_FR, _FC), jnp.inf, dtype=jnp.float32)

    def body(i, carry):
        lx, ly, lz, dists = carry
        dx = px - lx
        dy = py - ly
        dz = pz - lz
        d = (dx * dx + dz * dz) + dy * dy
        dists = jnp.minimum(dists, d)
        mx = jnp.max(dists)
        cand = jnp.where(dists == mx, idx, jnp.int32(N))
        nxt = jnp.min(cand)
        out_ref[i] = nxt
        rr = nxt // _FC
        cc = nxt - rr * _FC
        lm = jnp.where(lane1 == cc, jnp.float32(1.0), jnp.float32(0.0))
        nlx = jnp.sum(px_ref[pl.ds(rr, 1), :] * lm)
        nly = jnp.sum(py_ref[pl.ds(rr, 1), :] * lm)
        nlz = jnp.sum(pz_ref[pl.ds(rr, 1), :] * lm)
        return (nlx, nly, nlz, dists)

    jax.lax.fori_loop(1, M, body, (lx0, ly0, lz0, dists0))


def _fps(pos):
    px = pos[:, 0].reshape(_FR, _FC)
    py = pos[:, 1].reshape(_FR, _FC)
    pz = pos[:, 2].reshape(_FR, _FC)
    return pl.pallas_call(
        _fps_kernel,
        in_specs=[
            pl.BlockSpec(memory_space=pltpu.VMEM),
            pl.BlockSpec(memory_space=pltpu.VMEM),
            pl.BlockSpec(memory_space=pltpu.VMEM),
        ],
        out_specs=pl.BlockSpec(memory_space=pltpu.SMEM),
        out_shape=jax.ShapeDtypeStruct((M,), jnp.int32),
    )(px, py, pz)


def kernel(x, pos, batch, W, gamma, beta):
    id_clusters = _fps(pos)
    sub_pos = pos[id_clusters]
    sub_batch = batch[id_clusters]

    nn = _knn(pos, sub_pos)
    col = nn.reshape(-1)

    xp = jnp.concatenate([pos, x], axis=1)
    z = _compute_z(xp, W.T)
    zg = z[col]

    mx, s1, s1sum, s2sum = _segment_stats(zg)
    x_out = _finish(pos[:M], W[:, :3].T, mx, s1, s1sum, s2sum, gamma, beta)
    return (x_out, sub_pos, sub_batch)


# speedup vs baseline: 2.0817x; 1.0438x over previous
import functools

import jax
import jax.numpy as jnp
from jax.experimental import pallas as pl
from jax.experimental.pallas import tpu as pltpu
from jax.experimental.pallas import tpu_sc as plsc

N = 16384
IN_C = 64
OUT_C = 128
K = 16
M = 4096
MK = M * K


_KQ = 32
_KC = 128
_NCHUNK = N // _KC


def _bf(a):
    return a.astype(jnp.bfloat16).astype(jnp.float32)


_KU = 8


def _knn_kernel(posT_ref, pn_ref, spos_ref, qn_ref, out_ref, scr_ref):
    shape = (_KQ, _KC)
    lane = jax.lax.broadcasted_iota(jnp.int32, shape, 1)
    inf = jnp.float32(jnp.inf)

    qxb = jnp.broadcast_to(_bf(spos_ref[:, 0:1]), shape)
    qyb = jnp.broadcast_to(_bf(spos_ref[:, 1:2]), shape)
    qzb = jnp.broadcast_to(_bf(spos_ref[:, 2:3]), shape)
    qn = jnp.broadcast_to(qn_ref[...], shape)

    def fill_body(c, _):
        base = c * _KC
        xpc = _bf(posT_ref[0:1, pl.ds(base, _KC)])
        ypc = _bf(posT_ref[1:2, pl.ds(base, _KC)])
        zpc = _bf(posT_ref[2:3, pl.ds(base, _KC)])
        pnc = pn_ref[0:1, pl.ds(base, _KC)]
        v = (qn + pnc) - 2.0 * (qxb * xpc + qyb * ypc + qzb * zpc)
        scr_ref[:, pl.ds(base, _KC)] = v
        return 0

    jax.lax.fori_loop(0, _NCHUNK, fill_body, 0)

    pv = jnp.full(shape, -jnp.inf, dtype=jnp.float32)
    pi = jnp.full(shape, -1, dtype=jnp.int32)

    for r in range(K):
        def chunk_body(c, carry):
            av0, ai0, av1, ai1 = carry
            accs = [(av0, ai0), (av1, ai1)]
            for u in range(_KU):
                base = (c * _KU + u) * _KC
                v = scr_ref[:, pl.ds(base, _KC)]
                idx = lane + base
                elig = (v > pv) | ((v == pv) & (idx > pi))
                vm = jnp.where(elig, v, inf)
                accv, acci = accs[u % 2]
                take = vm < accv
                acci = jnp.where(take, idx, acci)
                accv = jnp.minimum(accv, vm)
                accs[u % 2] = (accv, acci)
            return (*accs[0], *accs[1])

        accv0 = jnp.full(shape, jnp.inf, dtype=jnp.float32)
        acci0 = jnp.zeros(shape, dtype=jnp.int32)
        av0, ai0, av1, ai1 = jax.lax.fori_loop(
            0, _NCHUNK // _KU, chunk_body, (accv0, acci0, accv0, acci0))
        take = (av1 < av0) | ((av1 == av0) & (ai1 < ai0))
        v = jnp.where(take, av1, av0)
        i = jnp.where(take, ai1, ai0)

        half = _KC // 2
        while half >= 1:
            lv, li = v[:, :half], i[:, :half]
            rv, ri = v[:, half:2 * half], i[:, half:2 * half]
            take = (rv < lv) | ((rv == lv) & (ri < li))
            v = jnp.where(take, rv, lv)
            i = jnp.where(take, ri, li)
            half //= 2
        out_ref[:, r:r + 1] = i
        pv = jnp.broadcast_to(v, shape)
        pi = jnp.broadcast_to(i, shape)


def _knn(pos, sub_pos):
    posT = pos.T
    pn = jnp.sum(pos ** 2, axis=1)[None, :]
    qn = jnp.sum(sub_pos ** 2, axis=1)[:, None]
    return pl.pallas_call(
        _knn_kernel,
        grid=(M // _KQ,),
        in_specs=[
            pl.BlockSpec((3, N), lambda i: (0, 0)),
            pl.BlockSpec((1, N), lambda i: (0, 0)),
            pl.BlockSpec((_KQ, 3), lambda i: (i, 0)),
            pl.BlockSpec((_KQ, 1), lambda i: (i, 0)),
        ],
        out_specs=pl.BlockSpec((_KQ, K), lambda i: (i, 0)),
        out_shape=jax.ShapeDtypeStruct((M, K), jnp.int32),
        scratch_shapes=[pltpu.VMEM((_KQ, N), jnp.float32)],
    )(posT, pn, sub_pos, qn)


def _z_kernel(xp_ref, wt_ref, z_ref):
    z_ref[...] = jax.lax.dot(xp_ref[...], wt_ref[...],
                             precision=jax.lax.Precision.HIGHEST)


def _compute_z(xp, wt):
    TR = 2048
    return pl.pallas_call(
        _z_kernel,
        grid=(N // TR,),
        in_specs=[
            pl.BlockSpec((TR, xp.shape[1]), lambda i: (i, 0)),
            pl.BlockSpec((xp.shape[1], OUT_C), lambda i: (0, 0)),
        ],
        out_specs=pl.BlockSpec((TR, OUT_C), lambda i: (i, 0)),
        out_shape=jax.ShapeDtypeStruct((N, OUT_C), jnp.float32),
    )(xp, wt)


_GW = 128


def _sc_gather(z, col):
    indices = col.reshape(1, MK)
    mesh = plsc.VectorSubcoreMesh(core_axis_name="core",
                                  subcore_axis_name="subcore")

    @functools.partial(
        pl.kernel,
        out_type=jax.ShapeDtypeStruct((MK, OUT_C), z.dtype),
        mesh=mesh)
    def kern(z_hbm, i_hbm, o_hbm):
        def body(i_vmem, o_vmem):
            pltpu.sync_copy(z_hbm.at[i_vmem.at[0]], o_vmem)

        pltpu.emit_pipeline(
            body,
            grid=(MK // _GW,),
            in_specs=[pl.BlockSpec((1, _GW), index_map=lambda i: (0, i))],
            out_specs=[pl.BlockSpec((_GW, OUT_C), index_map=lambda i: (i, 0))],
            core_axis_name="subcore",
            dimension_semantics=(pltpu.PARALLEL,),
        )(i_hbm, o_hbm)

    return kern(z, indices)


def _stats_kernel(zg_ref, mx_ref, s1_ref, s1sum_ref, s2sum_ref):
    step = pl.program_id(0)
    zt = zg_ref[...]
    z3 = zt.reshape(zt.shape[0] // K, K, OUT_C)
    s1 = z3.sum(axis=1)
    s2 = (z3 * z3).sum(axis=1)
    mx = z3.max(axis=1)
    mx_ref[...] = mx
    s1_ref[...] = s1
    ps1 = s1.sum(axis=0, keepdims=True)
    ps2 = s2.sum(axis=0, keepdims=True)

    @pl.when(step == 0)
    def _():
        s1sum_ref[...] = jnp.zeros_like(s1sum_ref)
        s2sum_ref[...] = jnp.zeros_like(s2sum_ref)

    s1sum_ref[...] += ps1
    s2sum_ref[...] += ps2


def _segment_stats(zg):
    TR = 4096
    SEG = TR // K
    return pl.pallas_call(
        _stats_kernel,
        grid=(MK // TR,),
        in_specs=[pl.BlockSpec((TR, OUT_C), lambda i: (i, 0))],
        out_specs=[
            pl.BlockSpec((SEG, OUT_C), lambda i: (i, 0)),
            pl.BlockSpec((SEG, OUT_C), lambda i: (i, 0)),
            pl.BlockSpec((1, OUT_C), lambda i: (0, 0)),
            pl.BlockSpec((1, OUT_C), lambda i: (0, 0)),
        ],
        out_shape=[
            jax.ShapeDtypeStruct((M, OUT_C), jnp.float32),
            jax.ShapeDtypeStruct((M, OUT_C), jnp.float32),
            jax.ShapeDtypeStruct((1, OUT_C), jnp.float32),
            jax.ShapeDtypeStruct((1, OUT_C), jnp.float32),
        ],
    )(zg)


def _finish_kernel(posm_ref, wpt_ref, mx_ref, s1_ref, s1sum_ref,
                   s2sum_ref, gamma_ref, beta_ref, out_ref):
    q = jax.lax.dot(posm_ref[...], wpt_ref[...],
                    precision=jax.lax.Precision.HIGHEST)
    s1 = s1_ref[...]
    qs = q.sum(axis=0, keepdims=True)
    mean = (s1sum_ref[...] - K * qs) / MK
    cross = (q * s1).sum(axis=0, keepdims=True)
    h2 = s2sum_ref[...] - 2.0 * cross + K * (q * q).sum(axis=0, keepdims=True)
    var = h2 / MK - mean * mean
    inv = jax.lax.rsqrt(var + 1e-5) * gamma_ref[...]
    out_ref[...] = jnp.maximum((mx_ref[...] - q - mean) * inv + beta_ref[...],
                               0.0)


def _finish(posm, wpt, mx, s1, s1sum, s2sum, gamma, beta):
    return pl.pallas_call(
        _finish_kernel,
        out_shape=jax.ShapeDtypeStruct((M, OUT_C), jnp.float32),
    )(posm, wpt, mx, s1, s1sum, s2sum, gamma.reshape(1, OUT_C),
      beta.reshape(1, OUT_C))


_FR = 128
_FC = N // _FR


def _fps_kernel(px_ref, py_ref, pz_ref, out_ref):
    px = px_ref[...]
    py = py_ref[...]
    pz = pz_ref[...]
    rows = jax.lax.broadcasted_iota(jnp.int32, (_FR, _FC), 0)
    cols = jax.lax.broadcasted_iota(jnp.int32, (_FR, _FC), 1)
    idx = rows * _FC + cols
    lane1 = jax.lax.broadcasted_iota(jnp.int32, (1, _FC), 1)
    out_ref[0] = 0
    lx0 = px[0, 0]
    ly0 = py[0, 0]
    lz0 = pz[0, 0]
    dists0 = jnp.full((_FR, _FC), jnp.inf, dtype=jnp.float32)

    def body(i, carry):
        lx, ly, lz, dists = carry
        dx = px - lx
        dy = py - ly
        dz = pz - lz
        d = (dx * dx + dz * dz) + dy * dy
        dists = jnp.minimum(dists, d)
        mx = jnp.max(dists)
        cand = jnp.where(dists == mx, idx, jnp.int32(N))
        nxt = jnp.min(cand)
        out_ref[i] = nxt
        rr = nxt // _FC
        cc = nxt - rr * _FC
        lm = jnp.where(lane1 == cc, jnp.float32(1.0), jnp.float32(0.0))
        nlx = jnp.sum(px_ref[pl.ds(rr, 1), :] * lm)
        nly = jnp.sum(py_ref[pl.ds(rr, 1), :] * lm)
        nlz = jnp.sum(pz_ref[pl.ds(rr, 1), :] * lm)
        return (nlx, nly, nlz, dists)

    jax.lax.fori_loop(1, M, body, (lx0, ly0, lz0, dists0))


def _fps(pos):
    px = pos[:, 0].reshape(_FR, _FC)
    py = pos[:, 1].reshape(_FR, _FC)
    pz = pos[:, 2].reshape(_FR, _FC)
    return pl.pallas_call(
        _fps_kernel,
        in_specs=[
            pl.BlockSpec(memory_space=pltpu.VMEM),
            pl.BlockSpec(memory_space=pltpu.VMEM),
            pl.BlockSpec(memory_space=pltpu.VMEM),
        ],
        out_specs=pl.BlockSpec(memory_space=pltpu.SMEM),
        out_shape=jax.ShapeDtypeStruct((M,), jnp.int32),
    )(px, py, pz)


def kernel(x, pos, batch, W, gamma, beta):
    id_clusters = _fps(pos)
    sub_pos = pos[id_clusters]
    sub_batch = batch[id_clusters]

    nn = _knn(pos, sub_pos)
    col = nn.reshape(-1)

    xp = jnp.concatenate([pos, x], axis=1)
    z = _compute_z(xp, W.T)
    zg = _sc_gather(z, col)

    mx, s1, s1sum, s2sum = _segment_stats(zg)
    x_out = _finish(pos[:M], W[:, :3].T, mx, s1, s1sum, s2sum, gamma, beta)
    return (x_out, sub_pos, sub_batch)
